# baseline (speedup 1.0000x reference)
import functools

import jax
import jax.numpy as jnp
from jax import lax
from jax.experimental import pallas as pl
from jax.experimental.pallas import tpu as pltpu
from jax.experimental.pallas import tpu_sc as plsc

BATCH = 16384
DIM = 32
ROWS = 1000000
NUM_CORES = 2
NUM_SUBCORES = 16
NW = NUM_CORES * NUM_SUBCORES
BPW = BATCH // NW
EPW = BPW * DIM
LANES = 16


def _gather_body(user_hbm, item_hbm, tug, tig, tum, tim,
                 gmf_o, eum_o, eim_o,
                 urows, irows, uidx, iidx, bug, big, bum, bim, sem):
    wid = lax.axis_index("s") * NUM_CORES + lax.axis_index("c")
    base = wid * BPW

    pltpu.sync_copy(user_hbm.at[pl.ds(base, BPW)], urows)
    pltpu.sync_copy(item_hbm.at[pl.ds(base, BPW)], irows)

    def idx_step(t, carry):
        sl = pl.ds(t * LANES, LANES)
        uv = urows[sl]
        iv = irows[sl]
        for c in range(DIM):
            dst = pl.ds(c * BPW + t * LANES, LANES)
            uidx[dst] = uv + (c * ROWS)
            iidx[dst] = iv + (c * ROWS)
        return carry

    lax.fori_loop(0, BPW // LANES, idx_step, 0)

    cs = [pltpu.async_copy(tug.at[uidx], bug, sem),
          pltpu.async_copy(tig.at[iidx], big, sem),
          pltpu.async_copy(tum.at[uidx], bum, sem),
          pltpu.async_copy(tim.at[iidx], bim, sem)]
    for c in cs:
        c.wait()

    def mul_step(t, carry):
        sl = pl.ds(t * LANES, LANES)
        bug[sl] = bug[sl] * big[sl]
        return carry

    lax.fori_loop(0, EPW // LANES, mul_step, 0, unroll=8)

    ws = []
    for c in range(DIM):
        src = pl.ds(c * BPW, BPW)
        dst = pl.ds(c * BATCH + base, BPW)
        ws.append(pltpu.async_copy(bug.at[src], gmf_o.at[dst], sem))
        ws.append(pltpu.async_copy(bum.at[src], eum_o.at[dst], sem))
        ws.append(pltpu.async_copy(bim.at[src], eim_o.at[dst], sem))
    for w in ws:
        w.wait()


_sc_gather = functools.partial(
    pl.kernel,
    out_type=[jax.ShapeDtypeStruct((DIM * BATCH,), jnp.float32)] * 3,
    mesh=plsc.VectorSubcoreMesh(core_axis_name="c", subcore_axis_name="s"),
    scratch_types=[
        pltpu.VMEM((BPW,), jnp.int32),
        pltpu.VMEM((BPW,), jnp.int32),
        pltpu.VMEM((EPW,), jnp.int32),
        pltpu.VMEM((EPW,), jnp.int32),
        pltpu.VMEM((EPW,), jnp.float32),
        pltpu.VMEM((EPW,), jnp.float32),
        pltpu.VMEM((EPW,), jnp.float32),
        pltpu.VMEM((EPW,), jnp.float32),
        pltpu.SemaphoreType.DMA,
    ],
)(_gather_body)


def _mlp_body(gmf, eum, eim, w1u, w1i, b1, w2, b2, w3, b3, wpm, wpg, bp, out):
    h = (jnp.dot(w1u[...], eum[...], preferred_element_type=jnp.float32)
         + jnp.dot(w1i[...], eim[...], preferred_element_type=jnp.float32)
         + b1[...])
    h = jnp.maximum(h, 0.0)
    h = jnp.maximum(jnp.dot(w2[...], h, preferred_element_type=jnp.float32) + b2[...], 0.0)
    h = jnp.maximum(jnp.dot(w3[...], h, preferred_element_type=jnp.float32) + b3[...], 0.0)
    p = (jnp.dot(wpm[...], h, preferred_element_type=jnp.float32)
         + jnp.dot(wpg[...], gmf[...], preferred_element_type=jnp.float32)
         + bp[...])
    out[...] = p


def kernel(user, item, embed_user_GMF, embed_item_GMF, embed_user_MLP, embed_item_MLP,
           W1, b1, W2, b2, W3, b3, Wp, bp):
    user = user.astype(jnp.int32)
    item = item.astype(jnp.int32)

    tug = embed_user_GMF.T.reshape(-1)
    tig = embed_item_GMF.T.reshape(-1)
    tum = embed_user_MLP.T.reshape(-1)
    tim = embed_item_MLP.T.reshape(-1)

    gmf_f, eum_f, eim_f = _sc_gather(user, item, tug, tig, tum, tim)
    gmf = gmf_f.reshape(DIM, BATCH)
    eum = eum_f.reshape(DIM, BATCH)
    eim = eim_f.reshape(DIM, BATCH)

    w1u = W1[:, :DIM]
    w1i = W1[:, DIM:]
    wpg = Wp[:, :DIM]
    wpm = Wp[:, DIM:]

    TB = 2048
    grid = (BATCH // TB,)
    act_spec = pl.BlockSpec((DIM, TB), lambda i: (0, i))
    full = lambda s: pl.BlockSpec(s, lambda i: (0,) * len(s))

    pred = pl.pallas_call(
        _mlp_body,
        grid=grid,
        in_specs=[
            act_spec, act_spec, act_spec,
            full(w1u.shape), full(w1i.shape), full((64, 1)),
            full(W2.shape), full((32, 1)),
            full(W3.shape), full((16, 1)),
            full(wpm.shape), full(wpg.shape), full((1, 1)),
        ],
        out_specs=pl.BlockSpec((1, TB), lambda i: (0, i)),
        out_shape=jax.ShapeDtypeStruct((1, BATCH), jnp.float32),
    )(gmf, eum, eim,
      w1u, w1i, b1.reshape(64, 1), W2, b2.reshape(32, 1),
      W3, b3.reshape(16, 1), wpm, wpg, bp.reshape(1, 1))

    return pred.reshape(-1)


# speedup vs baseline: 21.3602x; 21.3602x over previous
import functools

import jax
import jax.numpy as jnp
from jax import lax
from jax.experimental import pallas as pl
from jax.experimental.pallas import tpu as pltpu
from jax.experimental.pallas import tpu_sc as plsc

BATCH = 16384
DIM = 32
ROWS = 1000000
PACK = 4
PROW = 128
PROWS = ROWS // PACK
CB = 4096
NBLK = -(-ROWS // CB)
PROWS_PAD = NBLK * (CB // PACK)
NUM_CORES = 2
NUM_SUBCORES = 16
NW = NUM_CORES * NUM_SUBCORES
BPW = BATCH // NW
CHUNK = 128
NCHUNK = BPW // CHUNK
LANES = 16


def _repack_body(eye, t0, t1, t2, t3, o0, o1, o2, o3):
    lb = CB // PACK
    for t, o in ((t0, o0), (t1, o1), (t2, o2), (t3, o3)):
        x = t[...]
        tcat = jnp.concatenate([x[:, j * lb:(j + 1) * lb] for j in range(PACK)],
                               axis=0)
        o[...] = jax.lax.dot_general(tcat, eye[...], (((0,), (0,)), ((), ())),
                                     preferred_element_type=jnp.float32)


def _repack(tables):
    spec_in = pl.BlockSpec((DIM, CB), lambda i: (0, i))
    spec_out = pl.BlockSpec((CB // PACK, PROW), lambda i: (i, 0))
    eye = jnp.eye(PROW, dtype=jnp.float32)
    return pl.pallas_call(
        _repack_body,
        grid=(NBLK,),
        in_specs=[pl.BlockSpec((PROW, PROW), lambda i: (0, 0))] + [spec_in] * 4,
        out_specs=[spec_out] * 4,
        out_shape=[jax.ShapeDtypeStruct((PROWS_PAD, PROW), jnp.float32)] * 4,
    )(eye, *tables)


def _gather_body(user_hbm, item_hbm, tug, tig, tum, tim,
                 oug, oig, oum, oim,
                 uidx, iidx, bug, big, bum, bim, gsem):
    wid = lax.axis_index("s") * NUM_CORES + lax.axis_index("c")
    base = wid * BPW

    pltpu.sync_copy(user_hbm.at[pl.ds(base, BPW)], uidx)
    pltpu.sync_copy(item_hbm.at[pl.ds(base, BPW)], iidx)

    def shift_step(k, carry):
        sl = pl.ds(k * LANES, LANES)
        uv = uidx[sl]
        iv = iidx[sl]
        uidx[sl] = (lax.shift_right_logical(uv, 12) * 1024) + (uv & 1023)
        iidx[sl] = (lax.shift_right_logical(iv, 12) * 1024) + (iv & 1023)
        return carry

    lax.fori_loop(0, BPW // LANES, shift_step, 0, unroll=8)

    for j in range(NCHUNK):
        rows = pl.ds(j * CHUNK, CHUNK)
        cs = [pltpu.async_copy(tug.at[uidx.at[rows]], bug, gsem),
              pltpu.async_copy(tig.at[iidx.at[rows]], big, gsem),
              pltpu.async_copy(tum.at[uidx.at[rows]], bum, gsem),
              pltpu.async_copy(tim.at[iidx.at[rows]], bim, gsem)]
        for c in cs:
            c.wait()
        orows = pl.ds(base + j * CHUNK, CHUNK)
        pltpu.sync_copy(bug, oug.at[orows])
        pltpu.sync_copy(big, oig.at[orows])
        pltpu.sync_copy(bum, oum.at[orows])
        pltpu.sync_copy(bim, oim.at[orows])


_sc_gather = functools.partial(
    pl.kernel,
    out_type=[jax.ShapeDtypeStruct((BATCH, PROW), jnp.float32)] * 4,
    mesh=plsc.VectorSubcoreMesh(core_axis_name="c", subcore_axis_name="s"),
    scratch_types=[
        pltpu.VMEM((BPW,), jnp.int32),
        pltpu.VMEM((BPW,), jnp.int32),
        pltpu.VMEM((CHUNK, PROW), jnp.float32),
        pltpu.VMEM((CHUNK, PROW), jnp.float32),
        pltpu.VMEM((CHUNK, PROW), jnp.float32),
        pltpu.VMEM((CHUNK, PROW), jnp.float32),
        pltpu.SemaphoreType.DMA,
    ],
)(_gather_body)


def _extract(packed, sel):
    out = packed[:, 0:DIM]
    for k in range(1, PACK):
        out = jnp.where(sel == k, packed[:, k * DIM:(k + 1) * DIM], out)
    return out


def _mlp_body(uref, iref, pug, pig, pum, pim,
              w1u, w1i, b1, w2, b2, w3, b3, wpm, wpg, bp, out):
    usel = (lax.shift_right_logical(uref[0, 0, :], 10) & (PACK - 1)).reshape(-1, 1)
    isel = (lax.shift_right_logical(iref[0, 0, :], 10) & (PACK - 1)).reshape(-1, 1)
    eug = _extract(pug[...], usel)
    eig = _extract(pig[...], isel)
    eum = _extract(pum[...], usel)
    eim = _extract(pim[...], isel)
    gmf = eug * eig
    h = (jnp.dot(eum, w1u[...], preferred_element_type=jnp.float32)
         + jnp.dot(eim, w1i[...], preferred_element_type=jnp.float32)
         + b1[...])
    h = jnp.maximum(h, 0.0)
    h = jnp.maximum(jnp.dot(h, w2[...], preferred_element_type=jnp.float32) + b2[...], 0.0)
    h = jnp.maximum(jnp.dot(h, w3[...], preferred_element_type=jnp.float32) + b3[...], 0.0)
    p = (jnp.dot(h, wpm[...], preferred_element_type=jnp.float32)
         + jnp.dot(gmf, wpg[...], preferred_element_type=jnp.float32)
         + bp[...])
    out[...] = p


def kernel(user, item, embed_user_GMF, embed_item_GMF, embed_user_MLP, embed_item_MLP,
           W1, b1, W2, b2, W3, b3, Wp, bp):
    user = user.astype(jnp.int32)
    item = item.astype(jnp.int32)

    tables = (embed_user_GMF.T, embed_item_GMF.T, embed_user_MLP.T, embed_item_MLP.T)
    tug, tig, tum, tim = _repack(tables)

    pug, pig, pum, pim = _sc_gather(user, item, tug, tig, tum, tim)

    w1u = W1[:, :DIM].T
    w1i = W1[:, DIM:].T
    w2 = W2.T
    w3 = W3.T
    wpg = Wp[:, :DIM].T
    wpm = Wp[:, DIM:].T

    TB = 2048
    grid = (BATCH // TB,)
    row_spec = pl.BlockSpec((TB, PROW), lambda i: (i, 0))
    idx_spec = pl.BlockSpec((1, 1, TB), lambda i: (i, 0, 0))
    full = lambda s: pl.BlockSpec(s, lambda i: (0,) * len(s))

    pred = pl.pallas_call(
        _mlp_body,
        grid=grid,
        in_specs=[
            idx_spec, idx_spec,
            row_spec, row_spec, row_spec, row_spec,
            full(w1u.shape), full(w1i.shape), full((1, 64)),
            full(w2.shape), full((1, 32)),
            full(w3.shape), full((1, 16)),
            full(wpm.shape), full(wpg.shape), full((1, 1)),
        ],
        out_specs=pl.BlockSpec((TB, 1), lambda i: (i, 0)),
        out_shape=jax.ShapeDtypeStruct((BATCH, 1), jnp.float32),
    )(user.reshape(-1, 1, TB), item.reshape(-1, 1, TB),
      pug, pig, pum, pim,
      w1u, w1i, b1.reshape(1, 64), w2, b2.reshape(1, 32),
      w3, b3.reshape(1, 16), wpm, wpg, bp.reshape(1, 1))

    return pred.reshape(-1)


# speedup vs baseline: 24.6078x; 1.1520x over previous
import functools

import jax
import jax.numpy as jnp
from jax import lax
from jax.experimental import pallas as pl
from jax.experimental.pallas import tpu as pltpu
from jax.experimental.pallas import tpu_sc as plsc

BATCH = 16384
DIM = 32
ROWS = 1000000
PACK = 4
PROW = 128
PROWS = ROWS // PACK
CB = 16384
NBLK = -(-ROWS // CB)
PROWS_PAD = NBLK * (CB // PACK)
NUM_CORES = 2
NUM_SUBCORES = 16
NW = NUM_CORES * NUM_SUBCORES
BPW = BATCH // NW
CHUNK = 128
NCHUNK = BPW // CHUNK
LANES = 16


def _repack_body(eye, t0, t1, t2, t3, o0, o1, o2, o3):
    lb = CB // PACK
    for t, o in ((t0, o0), (t1, o1), (t2, o2), (t3, o3)):
        x = t[...]
        tcat = jnp.concatenate([x[:, j * lb:(j + 1) * lb] for j in range(PACK)],
                               axis=0)
        o[...] = jax.lax.dot_general(tcat, eye[...], (((0,), (0,)), ((), ())),
                                     preferred_element_type=jnp.float32)


def _repack(tables):
    spec_in = pl.BlockSpec((DIM, CB), lambda i: (0, i))
    spec_out = pl.BlockSpec((CB // PACK, PROW), lambda i: (i, 0))
    eye = jnp.eye(PROW, dtype=jnp.float32)
    return pl.pallas_call(
        _repack_body,
        grid=(NBLK,),
        in_specs=[pl.BlockSpec((PROW, PROW), lambda i: (0, 0))] + [spec_in] * 4,
        out_specs=[spec_out] * 4,
        out_shape=[jax.ShapeDtypeStruct((PROWS_PAD, PROW), jnp.float32)] * 4,
    )(eye, *tables)


def _gather_body(user_hbm, item_hbm, tug, tig, tum, tim,
                 oug, oig, oum, oim,
                 uidx, iidx, bug, big, bum, bim, gsem):
    wid = lax.axis_index("s") * NUM_CORES + lax.axis_index("c")
    base = wid * BPW

    pltpu.sync_copy(user_hbm.at[pl.ds(base, BPW)], uidx)
    pltpu.sync_copy(item_hbm.at[pl.ds(base, BPW)], iidx)

    def shift_step(k, carry):
        sl = pl.ds(k * LANES, LANES)
        uv = uidx[sl]
        iv = iidx[sl]
        uidx[sl] = (lax.shift_right_logical(uv, 14) * 4096) + (uv & 4095)
        iidx[sl] = (lax.shift_right_logical(iv, 14) * 4096) + (iv & 4095)
        return carry

    lax.fori_loop(0, BPW // LANES, shift_step, 0, unroll=8)

    for j in range(NCHUNK):
        rows = pl.ds(j * CHUNK, CHUNK)
        cs = [pltpu.async_copy(tug.at[uidx.at[rows]], bug, gsem),
              pltpu.async_copy(tig.at[iidx.at[rows]], big, gsem),
              pltpu.async_copy(tum.at[uidx.at[rows]], bum, gsem),
              pltpu.async_copy(tim.at[iidx.at[rows]], bim, gsem)]
        for c in cs:
            c.wait()
        orows = pl.ds(base + j * CHUNK, CHUNK)
        pltpu.sync_copy(bug, oug.at[orows])
        pltpu.sync_copy(big, oig.at[orows])
        pltpu.sync_copy(bum, oum.at[orows])
        pltpu.sync_copy(bim, oim.at[orows])


_sc_gather = functools.partial(
    pl.kernel,
    out_type=[jax.ShapeDtypeStruct((BATCH, PROW), jnp.float32)] * 4,
    mesh=plsc.VectorSubcoreMesh(core_axis_name="c", subcore_axis_name="s"),
    scratch_types=[
        pltpu.VMEM((BPW,), jnp.int32),
        pltpu.VMEM((BPW,), jnp.int32),
        pltpu.VMEM((CHUNK, PROW), jnp.float32),
        pltpu.VMEM((CHUNK, PROW), jnp.float32),
        pltpu.VMEM((CHUNK, PROW), jnp.float32),
        pltpu.VMEM((CHUNK, PROW), jnp.float32),
        pltpu.SemaphoreType.DMA,
    ],
)(_gather_body)


def _extract(packed, sel):
    out = packed[:, 0:DIM]
    for k in range(1, PACK):
        out = jnp.where(sel == k, packed[:, k * DIM:(k + 1) * DIM], out)
    return out


def _mlp_body(uref, iref, pug, pig, pum, pim,
              w1u, w1i, b1, w2, b2, w3, b3, wpm, wpg, bp, out):
    usel = (lax.shift_right_logical(uref[0, 0, :], 12) & (PACK - 1)).reshape(-1, 1)
    isel = (lax.shift_right_logical(iref[0, 0, :], 12) & (PACK - 1)).reshape(-1, 1)
    eug = _extract(pug[...], usel)
    eig = _extract(pig[...], isel)
    eum = _extract(pum[...], usel)
    eim = _extract(pim[...], isel)
    gmf = eug * eig
    h = (jnp.dot(eum, w1u[...], preferred_element_type=jnp.float32)
         + jnp.dot(eim, w1i[...], preferred_element_type=jnp.float32)
         + b1[...])
    h = jnp.maximum(h, 0.0)
    h = jnp.maximum(jnp.dot(h, w2[...], preferred_element_type=jnp.float32) + b2[...], 0.0)
    h = jnp.maximum(jnp.dot(h, w3[...], preferred_element_type=jnp.float32) + b3[...], 0.0)
    p = (jnp.dot(h, wpm[...], preferred_element_type=jnp.float32)
         + jnp.dot(gmf, wpg[...], preferred_element_type=jnp.float32)
         + bp[...])
    out[...] = p


def kernel(user, item, embed_user_GMF, embed_item_GMF, embed_user_MLP, embed_item_MLP,
           W1, b1, W2, b2, W3, b3, Wp, bp):
    user = user.astype(jnp.int32)
    item = item.astype(jnp.int32)

    tables = (embed_user_GMF.T, embed_item_GMF.T, embed_user_MLP.T, embed_item_MLP.T)
    tug, tig, tum, tim = _repack(tables)

    pug, pig, pum, pim = _sc_gather(user, item, tug, tig, tum, tim)

    w1u = W1[:, :DIM].T
    w1i = W1[:, DIM:].T
    w2 = W2.T
    w3 = W3.T
    wpg = Wp[:, :DIM].T
    wpm = Wp[:, DIM:].T

    TB = 4096
    grid = (BATCH // TB,)
    row_spec = pl.BlockSpec((TB, PROW), lambda i: (i, 0))
    idx_spec = pl.BlockSpec((1, 1, TB), lambda i: (i, 0, 0))
    full = lambda s: pl.BlockSpec(s, lambda i: (0,) * len(s))

    pred = pl.pallas_call(
        _mlp_body,
        grid=grid,
        in_specs=[
            idx_spec, idx_spec,
            row_spec, row_spec, row_spec, row_spec,
            full(w1u.shape), full(w1i.shape), full((1, 64)),
            full(w2.shape), full((1, 32)),
            full(w3.shape), full((1, 16)),
            full(wpm.shape), full(wpg.shape), full((1, 1)),
        ],
        out_specs=pl.BlockSpec((TB, 1), lambda i: (i, 0)),
        out_shape=jax.ShapeDtypeStruct((BATCH, 1), jnp.float32),
    )(user.reshape(-1, 1, TB), item.reshape(-1, 1, TB),
      pug, pig, pum, pim,
      w1u, w1i, b1.reshape(1, 64), w2, b2.reshape(1, 32),
      w3, b3.reshape(1, 16), wpm, wpg, bp.reshape(1, 1))

    return pred.reshape(-1)


# speedup vs baseline: 27.0372x; 1.0987x over previous
import functools

import jax
import jax.numpy as jnp
from jax import lax
from jax.experimental import pallas as pl
from jax.experimental.pallas import tpu as pltpu
from jax.experimental.pallas import tpu_sc as plsc

BATCH = 16384
DIM = 32
ROWS = 1000000
PACK = 8
PROW = 128
CB = 16384
LB = CB // PACK
NBLK = -(-ROWS // CB)
PROWS_PAD = NBLK * LB
NUM_CORES = 2
NUM_SUBCORES = 16
NW = NUM_CORES * NUM_SUBCORES
BPW = BATCH // NW
CHUNK = 128
NCHUNK = BPW // CHUNK
LANES = 16


def _repack_body(eye_e, eye_o, t0, t1, t2, t3, o0, o1, o2, o3):
    def half(tcat, eye):
        y = jax.lax.dot_general(tcat, eye, (((0,), (0,)), ((), ())),
                                preferred_element_type=jnp.float32)
        u16 = lax.bitcast_convert_type(y.astype(jnp.bfloat16), jnp.uint16)
        return u16.astype(jnp.uint32)

    for t, o in ((t0, o0), (t1, o1), (t2, o2), (t3, o3)):
        x = t[...]
        tcat = jnp.concatenate([x[:, j * LB:(j + 1) * LB] for j in range(PACK)],
                               axis=0)
        lo = half(tcat, eye_e[...])
        hi = half(tcat, eye_o[...]) << 16
        o[...] = lax.bitcast_convert_type(lo | hi, jnp.float32)


def _repack(tables):
    spec_in = pl.BlockSpec((DIM, CB), lambda i: (0, i))
    spec_out = pl.BlockSpec((LB, PROW), lambda i: (i, 0))
    k = jnp.arange(DIM * PACK)
    col = (k // DIM) * 16 + (k % DIM) // 2
    onehot = jax.nn.one_hot(col, PROW, dtype=jnp.float32)
    eye_e = onehot * ((k % 2) == 0)[:, None]
    eye_o = onehot * ((k % 2) == 1)[:, None]
    return pl.pallas_call(
        _repack_body,
        grid=(NBLK,),
        in_specs=[pl.BlockSpec((DIM * PACK, PROW), lambda i: (0, 0))] * 2
        + [spec_in] * 4,
        out_specs=[spec_out] * 4,
        out_shape=[jax.ShapeDtypeStruct((PROWS_PAD, PROW), jnp.float32)] * 4,
    )(eye_e, eye_o, *tables)


def _gather_body(user_hbm, item_hbm, tug, tig, tum, tim,
                 oug, oig, oum, oim,
                 uidx, iidx, bug, big, bum, bim, gsem):
    wid = lax.axis_index("s") * NUM_CORES + lax.axis_index("c")
    base = wid * BPW

    pltpu.sync_copy(user_hbm.at[pl.ds(base, BPW)], uidx)
    pltpu.sync_copy(item_hbm.at[pl.ds(base, BPW)], iidx)

    def shift_step(k, carry):
        sl = pl.ds(k * LANES, LANES)
        uv = uidx[sl]
        iv = iidx[sl]
        uidx[sl] = (lax.shift_right_logical(uv, 14) * LB) + (uv & (LB - 1))
        iidx[sl] = (lax.shift_right_logical(iv, 14) * LB) + (iv & (LB - 1))
        return carry

    lax.fori_loop(0, BPW // LANES, shift_step, 0, unroll=8)

    for j in range(NCHUNK):
        rows = pl.ds(j * CHUNK, CHUNK)
        cs = [pltpu.async_copy(tug.at[uidx.at[rows]], bug, gsem),
              pltpu.async_copy(tig.at[iidx.at[rows]], big, gsem),
              pltpu.async_copy(tum.at[uidx.at[rows]], bum, gsem),
              pltpu.async_copy(tim.at[iidx.at[rows]], bim, gsem)]
        for c in cs:
            c.wait()
        orows = pl.ds(base + j * CHUNK, CHUNK)
        pltpu.sync_copy(bug, oug.at[orows])
        pltpu.sync_copy(big, oig.at[orows])
        pltpu.sync_copy(bum, oum.at[orows])
        pltpu.sync_copy(bim, oim.at[orows])


_sc_gather = functools.partial(
    pl.kernel,
    out_type=[jax.ShapeDtypeStruct((BATCH, PROW), jnp.float32)] * 4,
    mesh=plsc.VectorSubcoreMesh(core_axis_name="c", subcore_axis_name="s"),
    scratch_types=[
        pltpu.VMEM((BPW,), jnp.int32),
        pltpu.VMEM((BPW,), jnp.int32),
        pltpu.VMEM((CHUNK, PROW), jnp.float32),
        pltpu.VMEM((CHUNK, PROW), jnp.float32),
        pltpu.VMEM((CHUNK, PROW), jnp.float32),
        pltpu.VMEM((CHUNK, PROW), jnp.float32),
        pltpu.SemaphoreType.DMA,
    ],
)(_gather_body)


def _extract(packed, sel):
    seg = packed[:, 0:16]
    for s in range(1, PACK):
        seg = jnp.where(sel == s, packed[:, s * 16:(s + 1) * 16], seg)
    u = lax.bitcast_convert_type(seg, jnp.uint32)
    even = lax.bitcast_convert_type(u << 16, jnp.float32)
    odd = lax.bitcast_convert_type(u & jnp.uint32(0xFFFF0000), jnp.float32)
    return even, odd


def _mlp_body(uref, iref, pug, pig, pum, pim,
              w1ue, w1uo, w1ie, w1io, b1, w2, b2, w3, b3, wpm, wpge, wpgo, bp,
              out):
    usel = (lax.shift_right_logical(uref[0, 0, :], 11) & (PACK - 1)).reshape(-1, 1)
    isel = (lax.shift_right_logical(iref[0, 0, :], 11) & (PACK - 1)).reshape(-1, 1)
    euge, eugo = _extract(pug[...], usel)
    eige, eigo = _extract(pig[...], isel)
    eume, eumo = _extract(pum[...], usel)
    eime, eimo = _extract(pim[...], isel)
    gmfe = euge * eige
    gmfo = eugo * eigo
    h = (jnp.dot(eume, w1ue[...], preferred_element_type=jnp.float32)
         + jnp.dot(eumo, w1uo[...], preferred_element_type=jnp.float32)
         + jnp.dot(eime, w1ie[...], preferred_element_type=jnp.float32)
         + jnp.dot(eimo, w1io[...], preferred_element_type=jnp.float32)
         + b1[...])
    h = jnp.maximum(h, 0.0)
    h = jnp.maximum(jnp.dot(h, w2[...], preferred_element_type=jnp.float32) + b2[...], 0.0)
    h = jnp.maximum(jnp.dot(h, w3[...], preferred_element_type=jnp.float32) + b3[...], 0.0)
    p = (jnp.dot(h, wpm[...], preferred_element_type=jnp.float32)
         + jnp.dot(gmfe, wpge[...], preferred_element_type=jnp.float32)
         + jnp.dot(gmfo, wpgo[...], preferred_element_type=jnp.float32)
         + bp[...])
    out[...] = p


def kernel(user, item, embed_user_GMF, embed_item_GMF, embed_user_MLP, embed_item_MLP,
           W1, b1, W2, b2, W3, b3, Wp, bp):
    user = user.astype(jnp.int32)
    item = item.astype(jnp.int32)

    tables = (embed_user_GMF.T, embed_item_GMF.T, embed_user_MLP.T, embed_item_MLP.T)
    tug, tig, tum, tim = _repack(tables)

    pug, pig, pum, pim = _sc_gather(user, item, tug, tig, tum, tim)

    w1u = W1[:, :DIM].T
    w1i = W1[:, DIM:].T
    w2 = W2.T
    w3 = W3.T
    wpg = Wp[:, :DIM].T
    wpm = Wp[:, DIM:].T

    TB = 4096
    grid = (BATCH // TB,)
    row_spec = pl.BlockSpec((TB, PROW), lambda i: (i, 0))
    idx_spec = pl.BlockSpec((1, 1, TB), lambda i: (i, 0, 0))
    full = lambda s: pl.BlockSpec(s, lambda i: (0,) * len(s))

    pred = pl.pallas_call(
        _mlp_body,
        grid=grid,
        in_specs=[
            idx_spec, idx_spec,
            row_spec, row_spec, row_spec, row_spec,
            full((16, 64)), full((16, 64)), full((16, 64)), full((16, 64)),
            full((1, 64)), full(w2.shape), full((1, 32)),
            full(w3.shape), full((1, 16)),
            full(wpm.shape), full((16, 1)), full((16, 1)), full((1, 1)),
        ],
        out_specs=pl.BlockSpec((TB, 1), lambda i: (i, 0)),
        out_shape=jax.ShapeDtypeStruct((BATCH, 1), jnp.float32),
    )(user.reshape(-1, 1, TB), item.reshape(-1, 1, TB),
      pug, pig, pum, pim,
      w1u[0::2], w1u[1::2], w1i[0::2], w1i[1::2],
      b1.reshape(1, 64), w2, b2.reshape(1, 32),
      w3, b3.reshape(1, 16), wpm, wpg[0::2], wpg[1::2], bp.reshape(1, 1))

    return pred.reshape(-1)


# speedup vs baseline: 27.1321x; 1.0035x over previous
import functools

import jax
import jax.numpy as jnp
from jax import lax
from jax.experimental import pallas as pl
from jax.experimental.pallas import tpu as pltpu
from jax.experimental.pallas import tpu_sc as plsc

BATCH = 16384
DIM = 32
ROWS = 1000000
PACK = 8
PROW = 128
CB = 16384
LB = CB // PACK
NBLK = -(-ROWS // CB)
PROWS_PAD = NBLK * LB
NUM_CORES = 2
NUM_SUBCORES = 16
NW = NUM_CORES * NUM_SUBCORES
BPW = BATCH // NW
CHUNK = 128
NCHUNK = BPW // CHUNK
LANES = 16


def _repack_body(eye_e, eye_o, t0, t1, t2, t3, o0, o1, o2, o3):
    def half(tcat, eye):
        y = jax.lax.dot_general(tcat, eye, (((0,), (0,)), ((), ())),
                                preferred_element_type=jnp.float32)
        u16 = lax.bitcast_convert_type(y.astype(jnp.bfloat16), jnp.uint16)
        return u16.astype(jnp.uint32)

    for t, o in ((t0, o0), (t1, o1), (t2, o2), (t3, o3)):
        x = t[...]
        tcat = jnp.concatenate([x[:, j * LB:(j + 1) * LB] for j in range(PACK)],
                               axis=0)
        lo = half(tcat, eye_e[...])
        hi = half(tcat, eye_o[...]) << 16
        o[...] = lax.bitcast_convert_type(lo | hi, jnp.float32)


def _repack(tables):
    spec_in = pl.BlockSpec((DIM, CB), lambda i: (0, i))
    spec_out = pl.BlockSpec((LB, PROW), lambda i: (i, 0))
    k = jnp.arange(DIM * PACK)
    col = (k // DIM) * 16 + (k % DIM) // 2
    onehot = jax.nn.one_hot(col, PROW, dtype=jnp.float32)
    eye_e = onehot * ((k % 2) == 0)[:, None]
    eye_o = onehot * ((k % 2) == 1)[:, None]
    return pl.pallas_call(
        _repack_body,
        grid=(NBLK,),
        in_specs=[pl.BlockSpec((DIM * PACK, PROW), lambda i: (0, 0))] * 2
        + [spec_in] * 4,
        out_specs=[spec_out] * 4,
        out_shape=[jax.ShapeDtypeStruct((PROWS_PAD, PROW), jnp.float32)] * 4,
    )(eye_e, eye_o, *tables)


def _gather_body(user_hbm, item_hbm, tug, tig, tum, tim,
                 oug, oig, oum, oim,
                 uidx, iidx, bug, big, bum, bim, gsem):
    wid = lax.axis_index("s") * NUM_CORES + lax.axis_index("c")
    base = wid * BPW

    pltpu.sync_copy(user_hbm.at[pl.ds(base, BPW)], uidx)
    pltpu.sync_copy(item_hbm.at[pl.ds(base, BPW)], iidx)

    def shift_step(k, carry):
        sl = pl.ds(k * LANES, LANES)
        uv = uidx[sl]
        iv = iidx[sl]
        uidx[sl] = (lax.shift_right_logical(uv, 14) * LB) + (uv & (LB - 1))
        iidx[sl] = (lax.shift_right_logical(iv, 14) * LB) + (iv & (LB - 1))
        return carry

    lax.fori_loop(0, BPW // LANES, shift_step, 0, unroll=8)

    for j in range(NCHUNK):
        rows = pl.ds(j * CHUNK, CHUNK)
        cs = [pltpu.async_copy(tug.at[uidx.at[rows]], bug, gsem),
              pltpu.async_copy(tig.at[iidx.at[rows]], big, gsem),
              pltpu.async_copy(tum.at[uidx.at[rows]], bum, gsem),
              pltpu.async_copy(tim.at[iidx.at[rows]], bim, gsem)]
        for c in cs:
            c.wait()
        orows = pl.ds(base + j * CHUNK, CHUNK)
        pltpu.sync_copy(bug, oug.at[orows])
        pltpu.sync_copy(big, oig.at[orows])
        pltpu.sync_copy(bum, oum.at[orows])
        pltpu.sync_copy(bim, oim.at[orows])


_sc_gather = functools.partial(
    pl.kernel,
    out_type=[jax.ShapeDtypeStruct((BATCH, PROW), jnp.float32)] * 4,
    mesh=plsc.VectorSubcoreMesh(core_axis_name="c", subcore_axis_name="s"),
    scratch_types=[
        pltpu.VMEM((BPW,), jnp.int32),
        pltpu.VMEM((BPW,), jnp.int32),
        pltpu.VMEM((CHUNK, PROW), jnp.float32),
        pltpu.VMEM((CHUNK, PROW), jnp.float32),
        pltpu.VMEM((CHUNK, PROW), jnp.float32),
        pltpu.VMEM((CHUNK, PROW), jnp.float32),
        pltpu.SemaphoreType.DMA,
    ],
)(_gather_body)


def _extract(packed, sel):
    seg = packed[:, 0:16]
    for s in range(1, PACK):
        seg = jnp.where(sel == s, packed[:, s * 16:(s + 1) * 16], seg)
    u = lax.bitcast_convert_type(seg, jnp.uint32)
    even = lax.bitcast_convert_type(u << 16, jnp.float32)
    odd = lax.bitcast_convert_type(u & jnp.uint32(0xFFFF0000), jnp.float32)
    return even, odd


def _mlp_body(uref, iref, pug, pig, pum, pim,
              w1cat, b1, w2, b2, w3, b3, wpm, wpgcat, bp,
              out):
    usel = (lax.shift_right_logical(uref[0, 0, :], 11) & (PACK - 1)).reshape(-1, 1)
    isel = (lax.shift_right_logical(iref[0, 0, :], 11) & (PACK - 1)).reshape(-1, 1)
    euge, eugo = _extract(pug[...], usel)
    eige, eigo = _extract(pig[...], isel)
    eume, eumo = _extract(pum[...], usel)
    eime, eimo = _extract(pim[...], isel)
    gmf = jnp.concatenate([euge * eige, eugo * eigo], axis=1)
    emlp = jnp.concatenate([eume, eumo, eime, eimo], axis=1)
    h = (jnp.dot(emlp, w1cat[...], preferred_element_type=jnp.float32)
         + b1[...])
    h = jnp.maximum(h, 0.0)
    h = jnp.maximum(jnp.dot(h, w2[...], preferred_element_type=jnp.float32) + b2[...], 0.0)
    h = jnp.maximum(jnp.dot(h, w3[...], preferred_element_type=jnp.float32) + b3[...], 0.0)
    p = (jnp.dot(h, wpm[...], preferred_element_type=jnp.float32)
         + jnp.dot(gmf, wpgcat[...], preferred_element_type=jnp.float32)
         + bp[...])
    out[...] = p


def kernel(user, item, embed_user_GMF, embed_item_GMF, embed_user_MLP, embed_item_MLP,
           W1, b1, W2, b2, W3, b3, Wp, bp):
    user = user.astype(jnp.int32)
    item = item.astype(jnp.int32)

    tables = (embed_user_GMF.T, embed_item_GMF.T, embed_user_MLP.T, embed_item_MLP.T)
    tug, tig, tum, tim = _repack(tables)

    pug, pig, pum, pim = _sc_gather(user, item, tug, tig, tum, tim)

    w1u = W1[:, :DIM].T
    w1i = W1[:, DIM:].T
    w2 = W2.T
    w3 = W3.T
    wpg = Wp[:, :DIM].T
    wpm = Wp[:, DIM:].T

    TB = 2048
    grid = (BATCH // TB,)
    row_spec = pl.BlockSpec((TB, PROW), lambda i: (i, 0))
    idx_spec = pl.BlockSpec((1, 1, TB), lambda i: (i, 0, 0))
    full = lambda s: pl.BlockSpec(s, lambda i: (0,) * len(s))

    pred = pl.pallas_call(
        _mlp_body,
        grid=grid,
        in_specs=[
            idx_spec, idx_spec,
            row_spec, row_spec, row_spec, row_spec,
            full((64, 64)),
            full((1, 64)), full(w2.shape), full((1, 32)),
            full(w3.shape), full((1, 16)),
            full(wpm.shape), full((32, 1)), full((1, 1)),
        ],
        out_specs=pl.BlockSpec((TB, 1), lambda i: (i, 0)),
        out_shape=jax.ShapeDtypeStruct((BATCH, 1), jnp.float32),
    )(user.reshape(-1, 1, TB), item.reshape(-1, 1, TB),
      pug, pig, pum, pim,
      jnp.concatenate([w1u[0::2], w1u[1::2], w1i[0::2], w1i[1::2]], axis=0),
      b1.reshape(1, 64), w2, b2.reshape(1, 32),
      w3, b3.reshape(1, 16), wpm,
      jnp.concatenate([wpg[0::2], wpg[1::2]], axis=0), bp.reshape(1, 1))

    return pred.reshape(-1)


# speedup vs baseline: 33.5839x; 1.2378x over previous
import functools

import jax
import jax.numpy as jnp
from jax import lax
from jax.experimental import pallas as pl
from jax.experimental.pallas import tpu as pltpu
from jax.experimental.pallas import tpu_sc as plsc

BATCH = 16384
DIM = 32
ROWS = 1000000
PACK = 8
PROW = 128
CB = 16384
LB = CB // PACK
NBLK = -(-ROWS // CB)
PROWS_PAD = NBLK * LB
NUM_CORES = 2
NUM_SUBCORES = 16
NW = NUM_CORES * NUM_SUBCORES
BPW = BATCH // NW
CHUNK = 128
NCHUNK = BPW // CHUNK
LANES = 16


def _repack_body(eye_e, eye_o, t0, t1, t2, t3, o0, o1, o2, o3):
    def half(tcat, eye):
        y = jax.lax.dot_general(tcat, eye, (((0,), (0,)), ((), ())),
                                preferred_element_type=jnp.float32)
        u16 = lax.bitcast_convert_type(y.astype(jnp.bfloat16), jnp.uint16)
        return u16.astype(jnp.uint32)

    for t, o in ((t0, o0), (t1, o1), (t2, o2), (t3, o3)):
        x = t[...]
        tcat = jnp.concatenate([x[:, j * LB:(j + 1) * LB] for j in range(PACK)],
                               axis=0)
        lo = half(tcat, eye_e[...])
        hi = half(tcat, eye_o[...]) << 16
        o[...] = lax.bitcast_convert_type(lo | hi, jnp.float32)


def _repack(tables):
    spec_in = pl.BlockSpec((DIM, CB), lambda i: (0, i))
    spec_out = pl.BlockSpec((LB, PROW), lambda i: (i, 0))
    k = jnp.arange(DIM * PACK)
    col = (k // DIM) * 16 + (k % DIM) // 2
    onehot = jax.nn.one_hot(col, PROW, dtype=jnp.float32)
    eye_e = onehot * ((k % 2) == 0)[:, None]
    eye_o = onehot * ((k % 2) == 1)[:, None]
    return pl.pallas_call(
        _repack_body,
        grid=(NBLK,),
        in_specs=[pl.BlockSpec((DIM * PACK, PROW), lambda i: (0, 0))] * 2
        + [spec_in] * 4,
        out_specs=[spec_out] * 4,
        out_shape=[jax.ShapeDtypeStruct((PROWS_PAD, PROW), jnp.float32)] * 4,
    )(eye_e, eye_o, *tables)


def _gather_body(user_hbm, item_hbm, tug, tig, tum, tim,
                 oug, oig, oum, oim,
                 uidx, iidx, ulane, ilane, bug, big, bum, bim, stg, gsem):
    wid = lax.axis_index("s") * NUM_CORES + lax.axis_index("c")
    base = wid * BPW

    pltpu.sync_copy(user_hbm.at[pl.ds(base, BPW)], uidx)
    pltpu.sync_copy(item_hbm.at[pl.ds(base, BPW)], iidx)

    def shift_step(k, carry):
        sl = pl.ds(k * LANES, LANES)
        uv = uidx[sl]
        iv = iidx[sl]
        ulane[sl] = (lax.shift_right_logical(uv, 11) & 7) * 16
        ilane[sl] = (lax.shift_right_logical(iv, 11) & 7) * 16
        uidx[sl] = (lax.shift_right_logical(uv, 14) * LB) + (uv & (LB - 1))
        iidx[sl] = (lax.shift_right_logical(iv, 14) * LB) + (iv & (LB - 1))
        return carry

    lax.fori_loop(0, BPW // LANES, shift_step, 0, unroll=8)

    lane_iota = lax.iota(jnp.int32, LANES)

    def extract_chunk(j, buf, lanebuf, out):
        def row_step(r, carry):
            rsplat = jnp.zeros((LANES,), jnp.int32) + r
            lb_ = plsc.load_gather(lanebuf, [rsplat + (j * CHUNK)])
            stg[pl.ds(r * 16, 16)] = plsc.load_gather(buf, [rsplat, lb_ + lane_iota])
            return carry

        lax.fori_loop(0, CHUNK, row_step, 0, unroll=4)
        pltpu.sync_copy(stg, out.at[pl.ds((base + j * CHUNK) * 16, CHUNK * 16)])

    for j in range(NCHUNK):
        rows = pl.ds(j * CHUNK, CHUNK)
        cs = [pltpu.async_copy(tug.at[uidx.at[rows]], bug, gsem),
              pltpu.async_copy(tig.at[iidx.at[rows]], big, gsem),
              pltpu.async_copy(tum.at[uidx.at[rows]], bum, gsem),
              pltpu.async_copy(tim.at[iidx.at[rows]], bim, gsem)]
        cs[0].wait()
        extract_chunk(j, bug, ulane, oug)
        cs[1].wait()
        extract_chunk(j, big, ilane, oig)
        cs[2].wait()
        extract_chunk(j, bum, ulane, oum)
        cs[3].wait()
        extract_chunk(j, bim, ilane, oim)


_sc_gather = functools.partial(
    pl.kernel,
    out_type=[jax.ShapeDtypeStruct((BATCH * 16,), jnp.float32)] * 4,
    mesh=plsc.VectorSubcoreMesh(core_axis_name="c", subcore_axis_name="s"),
    compiler_params=pltpu.CompilerParams(needs_layout_passes=False),
    scratch_types=[
        pltpu.VMEM((BPW,), jnp.int32),
        pltpu.VMEM((BPW,), jnp.int32),
        pltpu.VMEM((BPW,), jnp.int32),
        pltpu.VMEM((BPW,), jnp.int32),
        pltpu.VMEM((CHUNK, PROW), jnp.float32),
        pltpu.VMEM((CHUNK, PROW), jnp.float32),
        pltpu.VMEM((CHUNK, PROW), jnp.float32),
        pltpu.VMEM((CHUNK, PROW), jnp.float32),
        pltpu.VMEM((CHUNK * 16,), jnp.float32),
        pltpu.SemaphoreType.DMA,
    ],
)(_gather_body)


def _split(block):
    u = lax.bitcast_convert_type(block, jnp.uint32)
    even = lax.bitcast_convert_type(u << 16, jnp.float32)
    odd = lax.bitcast_convert_type(u & jnp.uint32(0xFFFF0000), jnp.float32)
    return even, odd


def _mlp_body(pug, pig, pum, pim, w1k, b1k, w2k, b2k, w3k, b3k, wpk, gpk, bp,
              out):
    euge, eugo = _split(pug[...])
    eige, eigo = _split(pig[...])
    eume, eumo = _split(pum[...])
    eime, eimo = _split(pim[...])
    gmf = jnp.concatenate([euge * eige, eugo * eigo], axis=1)
    emlp = jnp.concatenate([eume, eumo, eime, eimo], axis=1)
    h = jnp.dot(emlp, w1k[...], preferred_element_type=jnp.float32) + b1k[...]
    h = jnp.maximum(h, 0.0)
    h = jnp.maximum(jnp.dot(h, w2k[...], preferred_element_type=jnp.float32) + b2k[...], 0.0)
    h = jnp.maximum(jnp.dot(h, w3k[...], preferred_element_type=jnp.float32) + b3k[...], 0.0)
    p = (jnp.dot(h, wpk[...], preferred_element_type=jnp.float32)
         + jnp.dot(gmf, gpk[...], preferred_element_type=jnp.float32)
         + bp[...])
    out[...] = p


def kernel(user, item, embed_user_GMF, embed_item_GMF, embed_user_MLP, embed_item_MLP,
           W1, b1, W2, b2, W3, b3, Wp, bp):
    user = user.astype(jnp.int32)
    item = item.astype(jnp.int32)

    tables = (embed_user_GMF.T, embed_item_GMF.T, embed_user_MLP.T, embed_item_MLP.T)
    tug, tig, tum, tim = _repack(tables)

    pug, pig, pum, pim = _sc_gather(user, item, tug, tig, tum, tim)

    w1u = W1[:, :DIM].T
    w1i = W1[:, DIM:].T
    w2 = W2.T
    w3 = W3.T
    wpg = Wp[:, :DIM].T
    wpm = Wp[:, DIM:].T

    eye8 = jnp.eye(PACK, dtype=jnp.float32)
    kron = jnp.kron
    w1k = jnp.concatenate([kron(eye8, w1u[0::2]), kron(eye8, w1u[1::2]),
                           kron(eye8, w1i[0::2]), kron(eye8, w1i[1::2])],
                          axis=0)
    b1k = jnp.tile(b1, PACK).reshape(1, -1)
    w2k = kron(eye8, w2)
    b2k = jnp.tile(b2, PACK).reshape(1, -1)
    w3k = kron(eye8, w3)
    b3k = jnp.tile(b3, PACK).reshape(1, -1)
    wpk = kron(eye8, wpm)
    gpk = jnp.concatenate([kron(eye8, wpg[0::2]), kron(eye8, wpg[1::2])],
                          axis=0)

    QB = BATCH // PACK
    TQ = 512
    grid = (QB // TQ,)
    row_spec = pl.BlockSpec((TQ, PROW), lambda i: (i, 0))
    full = lambda s: pl.BlockSpec(s, lambda i: (0,) * len(s))

    pred = pl.pallas_call(
        _mlp_body,
        grid=grid,
        in_specs=[
            row_spec, row_spec, row_spec, row_spec,
            full((512, 512)), full((1, 512)),
            full((512, 256)), full((1, 256)),
            full((256, 128)), full((1, 128)),
            full((128, 8)), full((256, 8)), full((1, 1)),
        ],
        out_specs=pl.BlockSpec((TQ, 8), lambda i: (i, 0)),
        out_shape=jax.ShapeDtypeStruct((QB, 8), jnp.float32),
    )(pug.reshape(QB, PROW), pig.reshape(QB, PROW),
      pum.reshape(QB, PROW), pim.reshape(QB, PROW),
      w1k, b1k, w2k, b2k, w3k, b3k, wpk, gpk, bp.reshape(1, 1))

    return pred.reshape(-1)


# speedup vs baseline: 34.3947x; 1.0241x over previous
import functools

import jax
import jax.numpy as jnp
from jax import lax
from jax.experimental import pallas as pl
from jax.experimental.pallas import tpu as pltpu
from jax.experimental.pallas import tpu_sc as plsc

BATCH = 16384
DIM = 32
ROWS = 1000000
PACK = 8
PROW = 128
CB = 32768
LB = CB // PACK
NBLK = -(-ROWS // CB)
PROWS_PAD = NBLK * LB
NUM_CORES = 2
NUM_SUBCORES = 16
NW = NUM_CORES * NUM_SUBCORES
BPW = BATCH // NW
CHUNK = 128
NCHUNK = BPW // CHUNK
LANES = 16


def _repack_body(eye_e, eye_o, t0, t1, t2, t3, o0, o1, o2, o3):
    def half(tcat, eye):
        y = jax.lax.dot_general(tcat, eye, (((0,), (0,)), ((), ())),
                                preferred_element_type=jnp.float32)
        u16 = lax.bitcast_convert_type(y.astype(jnp.bfloat16), jnp.uint16)
        return u16.astype(jnp.uint32)

    for t, o in ((t0, o0), (t1, o1), (t2, o2), (t3, o3)):
        x = t[...]
        tcat = jnp.concatenate([x[:, j * LB:(j + 1) * LB] for j in range(PACK)],
                               axis=0)
        lo = half(tcat, eye_e[...])
        hi = half(tcat, eye_o[...]) << 16
        o[...] = lax.bitcast_convert_type(lo | hi, jnp.float32)


def _repack(tables):
    spec_in = pl.BlockSpec((DIM, CB), lambda i: (0, i))
    spec_out = pl.BlockSpec((LB, PROW), lambda i: (i, 0))
    k = jnp.arange(DIM * PACK)
    col = (k // DIM) * 16 + (k % DIM) // 2
    onehot = jax.nn.one_hot(col, PROW, dtype=jnp.float32)
    eye_e = onehot * ((k % 2) == 0)[:, None]
    eye_o = onehot * ((k % 2) == 1)[:, None]
    return pl.pallas_call(
        _repack_body,
        grid=(NBLK,),
        in_specs=[pl.BlockSpec((DIM * PACK, PROW), lambda i: (0, 0))] * 2
        + [spec_in] * 4,
        out_specs=[spec_out] * 4,
        out_shape=[jax.ShapeDtypeStruct((PROWS_PAD, PROW), jnp.float32)] * 4,
    )(eye_e, eye_o, *tables)


def _gather_body(user_hbm, item_hbm, tug, tig, tum, tim,
                 oug, oig, oum, oim,
                 uidx, iidx, ulane, ilane, bug, big, bum, bim, stg, gsem):
    wid = lax.axis_index("s") * NUM_CORES + lax.axis_index("c")
    base = wid * BPW

    pltpu.sync_copy(user_hbm.at[pl.ds(base, BPW)], uidx)
    pltpu.sync_copy(item_hbm.at[pl.ds(base, BPW)], iidx)

    def shift_step(k, carry):
        sl = pl.ds(k * LANES, LANES)
        uv = uidx[sl]
        iv = iidx[sl]
        ulane[sl] = (lax.shift_right_logical(uv, 12) & 7) * 16
        ilane[sl] = (lax.shift_right_logical(iv, 12) & 7) * 16
        uidx[sl] = (lax.shift_right_logical(uv, 15) * LB) + (uv & (LB - 1))
        iidx[sl] = (lax.shift_right_logical(iv, 15) * LB) + (iv & (LB - 1))
        return carry

    lax.fori_loop(0, BPW // LANES, shift_step, 0, unroll=8)

    lane_iota = lax.iota(jnp.int32, LANES)

    def extract_chunk(j, buf, lanebuf, out):
        def row_step(r, carry):
            rsplat = jnp.zeros((LANES,), jnp.int32) + r
            lb_ = plsc.load_gather(lanebuf, [rsplat + (j * CHUNK)])
            stg[pl.ds(r * 16, 16)] = plsc.load_gather(buf, [rsplat, lb_ + lane_iota])
            return carry

        lax.fori_loop(0, CHUNK, row_step, 0, unroll=4)
        pltpu.sync_copy(stg, out.at[pl.ds((base + j * CHUNK) * 16, CHUNK * 16)])

    for j in range(NCHUNK):
        rows = pl.ds(j * CHUNK, CHUNK)
        cs = [pltpu.async_copy(tug.at[uidx.at[rows]], bug, gsem),
              pltpu.async_copy(tig.at[iidx.at[rows]], big, gsem),
              pltpu.async_copy(tum.at[uidx.at[rows]], bum, gsem),
              pltpu.async_copy(tim.at[iidx.at[rows]], bim, gsem)]
        cs[0].wait()
        extract_chunk(j, bug, ulane, oug)
        cs[1].wait()
        extract_chunk(j, big, ilane, oig)
        cs[2].wait()
        extract_chunk(j, bum, ulane, oum)
        cs[3].wait()
        extract_chunk(j, bim, ilane, oim)


_sc_gather = functools.partial(
    pl.kernel,
    out_type=[jax.ShapeDtypeStruct((BATCH * 16,), jnp.float32)] * 4,
    mesh=plsc.VectorSubcoreMesh(core_axis_name="c", subcore_axis_name="s"),
    compiler_params=pltpu.CompilerParams(needs_layout_passes=False),
    scratch_types=[
        pltpu.VMEM((BPW,), jnp.int32),
        pltpu.VMEM((BPW,), jnp.int32),
        pltpu.VMEM((BPW,), jnp.int32),
        pltpu.VMEM((BPW,), jnp.int32),
        pltpu.VMEM((CHUNK, PROW), jnp.float32),
        pltpu.VMEM((CHUNK, PROW), jnp.float32),
        pltpu.VMEM((CHUNK, PROW), jnp.float32),
        pltpu.VMEM((CHUNK, PROW), jnp.float32),
        pltpu.VMEM((CHUNK * 16,), jnp.float32),
        pltpu.SemaphoreType.DMA,
    ],
)(_gather_body)


def _split(block):
    u = lax.bitcast_convert_type(block, jnp.uint32)
    even = lax.bitcast_convert_type(u << 16, jnp.float32)
    odd = lax.bitcast_convert_type(u & jnp.uint32(0xFFFF0000), jnp.float32)
    return even, odd


def _mlp_body(pug, pig, pum, pim, w1k, b1k, w2k, b2k, w3k, b3k, wpk, gpk, bp,
              out):
    euge, eugo = _split(pug[...])
    eige, eigo = _split(pig[...])
    eume, eumo = _split(pum[...])
    eime, eimo = _split(pim[...])
    gmf = jnp.concatenate([euge * eige, eugo * eigo], axis=1)
    emlp = jnp.concatenate([eume, eumo, eime, eimo], axis=1)
    h = jnp.dot(emlp, w1k[...], preferred_element_type=jnp.float32) + b1k[...]
    h = jnp.maximum(h, 0.0)
    h = jnp.maximum(jnp.dot(h, w2k[...], preferred_element_type=jnp.float32) + b2k[...], 0.0)
    h = jnp.maximum(jnp.dot(h, w3k[...], preferred_element_type=jnp.float32) + b3k[...], 0.0)
    p = (jnp.dot(h, wpk[...], preferred_element_type=jnp.float32)
         + jnp.dot(gmf, gpk[...], preferred_element_type=jnp.float32)
         + bp[...])
    out[...] = p


def kernel(user, item, embed_user_GMF, embed_item_GMF, embed_user_MLP, embed_item_MLP,
           W1, b1, W2, b2, W3, b3, Wp, bp):
    user = user.astype(jnp.int32)
    item = item.astype(jnp.int32)

    tables = (embed_user_GMF.T, embed_item_GMF.T, embed_user_MLP.T, embed_item_MLP.T)
    tug, tig, tum, tim = _repack(tables)

    pug, pig, pum, pim = _sc_gather(user, item, tug, tig, tum, tim)

    w1u = W1[:, :DIM].T
    w1i = W1[:, DIM:].T
    w2 = W2.T
    w3 = W3.T
    wpg = Wp[:, :DIM].T
    wpm = Wp[:, DIM:].T

    eye8 = jnp.eye(PACK, dtype=jnp.float32)
    kron = jnp.kron
    w1k = jnp.concatenate([kron(eye8, w1u[0::2]), kron(eye8, w1u[1::2]),
                           kron(eye8, w1i[0::2]), kron(eye8, w1i[1::2])],
                          axis=0)
    b1k = jnp.tile(b1, PACK).reshape(1, -1)
    w2k = kron(eye8, w2)
    b2k = jnp.tile(b2, PACK).reshape(1, -1)
    w3k = kron(eye8, w3)
    b3k = jnp.tile(b3, PACK).reshape(1, -1)
    wpk = kron(eye8, wpm)
    gpk = jnp.concatenate([kron(eye8, wpg[0::2]), kron(eye8, wpg[1::2])],
                          axis=0)

    QB = BATCH // PACK
    TQ = 512
    grid = (QB // TQ,)
    row_spec = pl.BlockSpec((TQ, PROW), lambda i: (i, 0))
    full = lambda s: pl.BlockSpec(s, lambda i: (0,) * len(s))

    pred = pl.pallas_call(
        _mlp_body,
        grid=grid,
        in_specs=[
            row_spec, row_spec, row_spec, row_spec,
            full((512, 512)), full((1, 512)),
            full((512, 256)), full((1, 256)),
            full((256, 128)), full((1, 128)),
            full((128, 8)), full((256, 8)), full((1, 1)),
        ],
        out_specs=pl.BlockSpec((TQ, 8), lambda i: (i, 0)),
        out_shape=jax.ShapeDtypeStruct((QB, 8), jnp.float32),
    )(pug.reshape(QB, PROW), pig.reshape(QB, PROW),
      pum.reshape(QB, PROW), pim.reshape(QB, PROW),
      w1k, b1k, w2k, b2k, w3k, b3k, wpk, gpk, bp.reshape(1, 1))

    return pred.reshape(-1)
